# baseline (speedup 1.0000x reference)
import functools

import jax
import jax.numpy as jnp
from jax.experimental import pallas as pl
from jax.experimental.pallas import tpu as pltpu

_UPDATE_RATE = 0.5
_AGE_FACTOR = 0.98


def _body(S, B, NS, M,
          hs_ref, mem0_ref,
          win_ref, wval_ref,
          wgx_ref, wgh_ref, wux_ref, wuh_ref, wrx_ref, wrh_ref,
          bin_ref, bval_ref, bg_ref, bu_ref, br_ref,
          out_ref,
          min_scr, xg_scr, xu_scr, xr_scr):
    f32 = jnp.float32
    bf16 = jnp.bfloat16

    hs = hs_ref[...].astype(bf16)
    m_in_all = jnp.dot(hs, win_ref[...].astype(bf16), preferred_element_type=f32) + bin_ref[...]
    vals = jnp.dot(hs, wval_ref[...].astype(bf16), preferred_element_type=f32) + bval_ref[...]
    vals16 = vals.astype(bf16)
    xg_all = jnp.dot(vals16, wgx_ref[...].astype(bf16), preferred_element_type=f32) + bg_ref[...]
    xu_all = jnp.dot(vals16, wux_ref[...].astype(bf16), preferred_element_type=f32) + bu_ref[...]
    xr_all = jnp.dot(vals16, wrx_ref[...].astype(bf16), preferred_element_type=f32) + br_ref[...]
    min_scr[...] = m_in_all.reshape(S, B, M)
    xg_scr[...] = xg_all.reshape(S, B, M)
    xu_scr[...] = xu_all.reshape(S, B, M)
    xr_scr[...] = xr_all.reshape(S, B, M)

    wgh = wgh_ref[...].astype(bf16)
    wuh = wuh_ref[...].astype(bf16)
    wrh = wrh_ref[...].astype(bf16)

    def step(t, carry):
        mem, usage, age = carry
        m_in = min_scr[t]
        xg = xg_scr[t]
        xu = xu_scr[t]
        xr = xr_scr[t]

        sim = jnp.sum(mem * m_in[:, None, :], axis=2)
        scores = usage * 0.2 + age * 0.1 - sim
        w = scores - jnp.max(scores, axis=1, keepdims=True)
        e = jnp.exp(w)
        write_w = e / jnp.sum(e, axis=1, keepdims=True)

        mem2 = mem.reshape(B * NS, M).astype(jnp.bfloat16)
        reset = jax.nn.sigmoid(
            jnp.dot(mem2, wrh, preferred_element_type=f32).reshape(B, NS, M)
            + xr[:, None, :])
        upd = jax.nn.sigmoid(
            jnp.dot(mem2, wgh, preferred_element_type=f32).reshape(B, NS, M)
            + xg[:, None, :])
        rh = (reset * mem).reshape(B * NS, M).astype(jnp.bfloat16)
        cand = jnp.tanh(
            jnp.dot(rh, wuh, preferred_element_type=f32).reshape(B, NS, M)
            + xu[:, None, :])
        new_h = (1.0 - upd) * mem + upd * cand

        ww = write_w[:, :, None] * _UPDATE_RATE
        updated = mem * (1.0 - ww) + new_h * ww
        mask = write_w > 0.01
        memn = jnp.where(mask[:, :, None], updated, mem)
        usage = usage + jnp.where(mask, write_w, jnp.zeros_like(write_w))
        norm = jnp.sqrt(jnp.sum(memn * memn, axis=2, keepdims=True))
        memn = memn / jnp.maximum(norm, 1e-12)
        age = age * _AGE_FACTOR + 1.0
        usage = usage * 0.99
        return memn, usage, age

    zeros = jnp.zeros((B, NS), dtype=f32)
    mem_final, _, _ = jax.lax.fori_loop(0, S, step, (mem0_ref[...], zeros, zeros))
    out_ref[...] = mem_final


@jax.jit
def kernel(hidden_states, memory0, W_in, b_in, W_val, b_val,
           W_gate, b_gate, W_upd, b_upd, W_reset, b_reset):
    B, S, D = hidden_states.shape
    _, NS, M = memory0.shape

    hs = jnp.transpose(hidden_states, (1, 0, 2)).reshape(S * B, D)
    win_t = W_in.T
    wval_t = W_val.T
    wgx, wgh = W_gate[:, :M].T, W_gate[:, M:].T
    wux, wuh = W_upd[:, :M].T, W_upd[:, M:].T
    wrx, wrh = W_reset[:, :M].T, W_reset[:, M:].T

    body = functools.partial(_body, S, B, NS, M)
    out = pl.pallas_call(
        body,
        out_shape=jax.ShapeDtypeStruct((B, NS, M), jnp.float32),
        scratch_shapes=[pltpu.VMEM((S, B, M), jnp.float32)] * 4,
    )(hs, memory0,
      win_t, wval_t, wgx, wgh, wux, wuh, wrx, wrh,
      b_in.reshape(1, M), b_val.reshape(1, M), b_gate.reshape(1, M),
      b_upd.reshape(1, M), b_reset.reshape(1, M))
    return out


# speedup vs baseline: 1.0687x; 1.0687x over previous
import functools

import jax
import jax.numpy as jnp
from jax.experimental import pallas as pl
from jax.experimental.pallas import tpu as pltpu

_UPDATE_RATE = 0.5
_AGE_FACTOR = 0.98


def _body(S, B, NS, M,
          hs_ref, mem0_ref,
          win_ref, wval_ref,
          wgx_ref, wgh_ref, wux_ref, wuh_ref, wrx_ref, wrh_ref,
          bin_ref, bval_ref, bg_ref, bu_ref, br_ref,
          out_ref,
          min_scr, xg_scr, xu_scr, xr_scr):
    f32 = jnp.float32

    hs = hs_ref[...]
    m_in_all = jnp.dot(hs, win_ref[...], preferred_element_type=f32) + bin_ref[...]
    vals = jnp.dot(hs, wval_ref[...], preferred_element_type=f32) + bval_ref[...]
    xg_all = jnp.dot(vals, wgx_ref[...], preferred_element_type=f32) + bg_ref[...]
    xu_all = jnp.dot(vals, wux_ref[...], preferred_element_type=f32) + bu_ref[...]
    xr_all = jnp.dot(vals, wrx_ref[...], preferred_element_type=f32) + br_ref[...]
    min_scr[...] = m_in_all.reshape(S, B, M)
    xg_scr[...] = xg_all.reshape(S, B, M)
    xu_scr[...] = xu_all.reshape(S, B, M)
    xr_scr[...] = xr_all.reshape(S, B, M)

    wgh = wgh_ref[...]
    wuh = wuh_ref[...]
    wrh = wrh_ref[...]

    H = B // 2

    def gru_chunk(memC, xrC, xgC, xuC, ww3C):
        mem2 = memC.reshape(H * NS, M)
        reset = jax.nn.sigmoid(
            jnp.dot(mem2, wrh, preferred_element_type=f32).reshape(H, NS, M)
            + xrC[:, None, :])
        upd = jax.nn.sigmoid(
            jnp.dot(mem2, wgh, preferred_element_type=f32).reshape(H, NS, M)
            + xgC[:, None, :])
        rh = (reset * memC).reshape(H * NS, M)
        cand = jnp.tanh(
            jnp.dot(rh, wuh, preferred_element_type=f32).reshape(H, NS, M)
            + xuC[:, None, :])
        new_h = (1.0 - upd) * memC + upd * cand
        updated = memC * (1.0 - ww3C) + new_h * ww3C
        norm = jnp.sqrt(jnp.sum(updated * updated, axis=2, keepdims=True))
        return updated / jnp.maximum(norm, 1e-12)

    def step(t, carry):
        memA, memB, usage, age = carry
        m_in = min_scr[t]
        xg = xg_scr[t]
        xu = xu_scr[t]
        xr = xr_scr[t]

        simA = jnp.sum(memA * m_in[:H, None, :], axis=2)
        simB = jnp.sum(memB * m_in[H:, None, :], axis=2)
        sim = jnp.concatenate([simA, simB], axis=0)
        scores = usage * 0.2 + age * 0.1 - sim
        w = scores - jnp.max(scores, axis=1, keepdims=True)
        e = jnp.exp(w)
        write_w = e / jnp.sum(e, axis=1, keepdims=True)
        wwm = jnp.where(write_w > 0.01, write_w, jnp.zeros_like(write_w))
        ww3 = (wwm * _UPDATE_RATE)[:, :, None]

        memnA = gru_chunk(memA, xr[:H], xg[:H], xu[:H], ww3[:H])
        memnB = gru_chunk(memB, xr[H:], xg[H:], xu[H:], ww3[H:])

        usage = (usage + wwm) * 0.99
        age = age * _AGE_FACTOR + 1.0
        return memnA, memnB, usage, age

    zeros = jnp.zeros((B, NS), dtype=f32)
    memA, memB, _, _ = jax.lax.fori_loop(
        0, S, step, (mem0_ref[:B // 2], mem0_ref[B // 2:], zeros, zeros))
    out_ref[:B // 2] = memA
    out_ref[B // 2:] = memB


@jax.jit
def kernel(hidden_states, memory0, W_in, b_in, W_val, b_val,
           W_gate, b_gate, W_upd, b_upd, W_reset, b_reset):
    B, S, D = hidden_states.shape
    _, NS, M = memory0.shape

    hs = jnp.transpose(hidden_states, (1, 0, 2)).reshape(S * B, D)
    win_t = W_in.T
    wval_t = W_val.T
    wgx, wgh = W_gate[:, :M].T, W_gate[:, M:].T
    wux, wuh = W_upd[:, :M].T, W_upd[:, M:].T
    wrx, wrh = W_reset[:, :M].T, W_reset[:, M:].T

    body = functools.partial(_body, S, B, NS, M)
    out = pl.pallas_call(
        body,
        out_shape=jax.ShapeDtypeStruct((B, NS, M), jnp.float32),
        scratch_shapes=[pltpu.VMEM((S, B, M), jnp.float32)] * 4,
    )(hs, memory0,
      win_t, wval_t, wgx, wgh, wux, wuh, wrx, wrh,
      b_in.reshape(1, M), b_val.reshape(1, M), b_gate.reshape(1, M),
      b_upd.reshape(1, M), b_reset.reshape(1, M))
    return out


# speedup vs baseline: 1.1709x; 1.0956x over previous
import functools

import jax
import jax.numpy as jnp
from jax.experimental import pallas as pl
from jax.experimental.pallas import tpu as pltpu

_UPDATE_RATE = 0.5
_AGE_FACTOR = 0.98


def _body(S, B, NS, M,
          hs_ref, mem0_ref,
          win_ref, wval_ref,
          wgx_ref, wgh_ref, wux_ref, wuh_ref, wrx_ref, wrh_ref,
          bin_ref, bval_ref, bg_ref, bu_ref, br_ref,
          out_ref,
          min_scr, xg_scr, xu_scr, xr_scr):
    f32 = jnp.float32

    hs = hs_ref[...]
    m_in_all = jnp.dot(hs, win_ref[...], preferred_element_type=f32) + bin_ref[...]
    vals = jnp.dot(hs, wval_ref[...], preferred_element_type=f32) + bval_ref[...]
    xg_all = jnp.dot(vals, wgx_ref[...], preferred_element_type=f32) + bg_ref[...]
    xu_all = jnp.dot(vals, wux_ref[...], preferred_element_type=f32) + bu_ref[...]
    xr_all = jnp.dot(vals, wrx_ref[...], preferred_element_type=f32) + br_ref[...]
    min_scr[...] = m_in_all.reshape(S, B, M)
    xg_scr[...] = xg_all.reshape(S, B, M)
    xu_scr[...] = xu_all.reshape(S, B, M)
    xr_scr[...] = xr_all.reshape(S, B, M)

    wgh = wgh_ref[...]
    wuh = wuh_ref[...]
    wrh = wrh_ref[...]

    H = B // 2

    def gru_chunk(memC, xrC, xgC, xuC, ww3C):
        mem2 = memC.reshape(H * NS, M)
        reset = jax.nn.sigmoid(
            jnp.dot(mem2, wrh, preferred_element_type=f32).reshape(H, NS, M)
            + xrC[:, None, :])
        upd = jax.nn.sigmoid(
            jnp.dot(mem2, wgh, preferred_element_type=f32).reshape(H, NS, M)
            + xgC[:, None, :])
        rh = (reset * memC).reshape(H * NS, M)
        cand = jnp.tanh(
            jnp.dot(rh, wuh, preferred_element_type=f32).reshape(H, NS, M)
            + xuC[:, None, :])
        new_h = (1.0 - upd) * memC + upd * cand
        updated = memC * (1.0 - ww3C) + new_h * ww3C
        norm = jnp.sqrt(jnp.sum(updated * updated, axis=2, keepdims=True))
        return updated / jnp.maximum(norm, 1e-12)

    def step(t, carry):
        memA, memB, usage, age = carry
        m_in = min_scr[t]
        xg = xg_scr[t]
        xu = xu_scr[t]
        xr = xr_scr[t]

        simA = jnp.sum(memA * m_in[:H, None, :], axis=2)
        simB = jnp.sum(memB * m_in[H:, None, :], axis=2)
        sim = jnp.concatenate([simA, simB], axis=0)
        scores = usage * 0.2 + age * 0.1 - sim
        w = scores - jnp.max(scores, axis=1, keepdims=True)
        e = jnp.exp(w)
        write_w = e / jnp.sum(e, axis=1, keepdims=True)
        wwm = jnp.where(write_w > 0.01, write_w, jnp.zeros_like(write_w))
        ww3 = (wwm * _UPDATE_RATE)[:, :, None]

        memnA = gru_chunk(memA, xr[:H], xg[:H], xu[:H], ww3[:H])
        memnB = gru_chunk(memB, xr[H:], xg[H:], xu[H:], ww3[H:])

        usage = (usage + wwm) * 0.99
        age = age * _AGE_FACTOR + 1.0
        return memnA, memnB, usage, age

    zeros = jnp.zeros((B, NS), dtype=f32)
    memA, memB, _, _ = jax.lax.fori_loop(
        0, S, step, (mem0_ref[:B // 2], mem0_ref[B // 2:], zeros, zeros),
        unroll=2)
    out_ref[:B // 2] = memA
    out_ref[B // 2:] = memB


@jax.jit
def kernel(hidden_states, memory0, W_in, b_in, W_val, b_val,
           W_gate, b_gate, W_upd, b_upd, W_reset, b_reset):
    B, S, D = hidden_states.shape
    _, NS, M = memory0.shape

    hs = jnp.transpose(hidden_states, (1, 0, 2)).reshape(S * B, D)
    win_t = W_in.T
    wval_t = W_val.T
    wgx, wgh = W_gate[:, :M].T, W_gate[:, M:].T
    wux, wuh = W_upd[:, :M].T, W_upd[:, M:].T
    wrx, wrh = W_reset[:, :M].T, W_reset[:, M:].T

    body = functools.partial(_body, S, B, NS, M)
    out = pl.pallas_call(
        body,
        out_shape=jax.ShapeDtypeStruct((B, NS, M), jnp.float32),
        scratch_shapes=[pltpu.VMEM((S, B, M), jnp.float32)] * 4,
    )(hs, memory0,
      win_t, wval_t, wgx, wgh, wux, wuh, wrx, wrh,
      b_in.reshape(1, M), b_val.reshape(1, M), b_gate.reshape(1, M),
      b_upd.reshape(1, M), b_reset.reshape(1, M))
    return out


# speedup vs baseline: 1.1731x; 1.0018x over previous
import functools

import jax
import jax.numpy as jnp
from jax.experimental import pallas as pl
from jax.experimental.pallas import tpu as pltpu

_UPDATE_RATE = 0.5
_AGE_FACTOR = 0.98


def _body(S, B, NS, M,
          hs_ref, mem0_ref,
          win_ref, wval_ref,
          wgx_ref, wgh_ref, wux_ref, wuh_ref, wrx_ref, wrh_ref,
          bin_ref, bval_ref, bg_ref, bu_ref, br_ref,
          out_ref,
          min_scr, xg_scr, xu_scr, xr_scr):
    f32 = jnp.float32

    hs = hs_ref[...]
    m_in_all = jnp.dot(hs, win_ref[...], preferred_element_type=f32) + bin_ref[...]
    vals = jnp.dot(hs, wval_ref[...], preferred_element_type=f32) + bval_ref[...]
    xg_all = jnp.dot(vals, wgx_ref[...], preferred_element_type=f32) + bg_ref[...]
    xu_all = jnp.dot(vals, wux_ref[...], preferred_element_type=f32) + bu_ref[...]
    xr_all = jnp.dot(vals, wrx_ref[...], preferred_element_type=f32) + br_ref[...]
    min_scr[...] = m_in_all.reshape(S, B, M)
    xg_scr[...] = xg_all.reshape(S, B, M)
    xu_scr[...] = xu_all.reshape(S, B, M)
    xr_scr[...] = xr_all.reshape(S, B, M)

    wgh = wgh_ref[...]
    wuh = wuh_ref[...]
    wrh = wrh_ref[...]

    H = B // 2

    def gru_chunk(memC, xrC, xgC, xuC, ww3C):
        mem2 = memC.reshape(H * NS, M)
        reset = jax.nn.sigmoid(
            jnp.dot(mem2, wrh, preferred_element_type=f32).reshape(H, NS, M)
            + xrC[:, None, :])
        upd = jax.nn.sigmoid(
            jnp.dot(mem2, wgh, preferred_element_type=f32).reshape(H, NS, M)
            + xgC[:, None, :])
        rh = (reset * memC).reshape(H * NS, M)
        cand = jnp.tanh(
            jnp.dot(rh, wuh, preferred_element_type=f32).reshape(H, NS, M)
            + xuC[:, None, :])
        new_h = (1.0 - upd) * memC + upd * cand
        updated = memC * (1.0 - ww3C) + new_h * ww3C
        norm = jnp.sqrt(jnp.sum(updated * updated, axis=2, keepdims=True))
        return updated / jnp.maximum(norm, 1e-12)

    def step(t, carry):
        memA, memB, usage, age = carry
        m_in = min_scr[t]
        xg = xg_scr[t]
        xu = xu_scr[t]
        xr = xr_scr[t]

        simA = jnp.sum(memA * m_in[:H, None, :], axis=2)
        simB = jnp.sum(memB * m_in[H:, None, :], axis=2)
        sim = jnp.concatenate([simA, simB], axis=0)
        scores = usage * 0.2 + age * 0.1 - sim
        w = scores - jnp.max(scores, axis=1, keepdims=True)
        e = jnp.exp(w)
        write_w = e / jnp.sum(e, axis=1, keepdims=True)
        wwm = jnp.where(write_w > 0.01, write_w, jnp.zeros_like(write_w))
        ww3 = (wwm * _UPDATE_RATE)[:, :, None]

        memnA = gru_chunk(memA, xr[:H], xg[:H], xu[:H], ww3[:H])
        memnB = gru_chunk(memB, xr[H:], xg[H:], xu[H:], ww3[H:])

        usage = (usage + wwm) * 0.99
        age = age * _AGE_FACTOR + 1.0
        return memnA, memnB, usage, age

    zeros = jnp.zeros((B, NS), dtype=f32)
    memA, memB, _, _ = jax.lax.fori_loop(
        0, S, step, (mem0_ref[:B // 2], mem0_ref[B // 2:], zeros, zeros),
        unroll=4)
    out_ref[:B // 2] = memA
    out_ref[B // 2:] = memB


@jax.jit
def kernel(hidden_states, memory0, W_in, b_in, W_val, b_val,
           W_gate, b_gate, W_upd, b_upd, W_reset, b_reset):
    B, S, D = hidden_states.shape
    _, NS, M = memory0.shape

    hs = jnp.transpose(hidden_states, (1, 0, 2)).reshape(S * B, D)
    win_t = W_in.T
    wval_t = W_val.T
    wgx, wgh = W_gate[:, :M].T, W_gate[:, M:].T
    wux, wuh = W_upd[:, :M].T, W_upd[:, M:].T
    wrx, wrh = W_reset[:, :M].T, W_reset[:, M:].T

    body = functools.partial(_body, S, B, NS, M)
    out = pl.pallas_call(
        body,
        out_shape=jax.ShapeDtypeStruct((B, NS, M), jnp.float32),
        scratch_shapes=[pltpu.VMEM((S, B, M), jnp.float32)] * 4,
    )(hs, memory0,
      win_t, wval_t, wgx, wgh, wux, wuh, wrx, wrh,
      b_in.reshape(1, M), b_val.reshape(1, M), b_gate.reshape(1, M),
      b_upd.reshape(1, M), b_reset.reshape(1, M))
    return out


# speedup vs baseline: 1.1899x; 1.0143x over previous
import functools

import jax
import jax.numpy as jnp
from jax.experimental import pallas as pl
from jax.experimental.pallas import tpu as pltpu

_UPDATE_RATE = 0.5
_AGE_FACTOR = 0.98


def _body(S, B, NS, M,
          hs_ref, mem0_ref,
          win_ref, wval_ref,
          wgx_ref, wgh_ref, wux_ref, wuh_ref, wrx_ref, wrh_ref,
          bin_ref, bval_ref, bg_ref, bu_ref, br_ref,
          out_ref,
          min_scr, xg_scr, xu_scr, xr_scr):
    f32 = jnp.float32

    hs = hs_ref[...]
    m_in_all = jnp.dot(hs, win_ref[...], preferred_element_type=f32) + bin_ref[...]
    vals = jnp.dot(hs, wval_ref[...], preferred_element_type=f32) + bval_ref[...]
    xg_all = jnp.dot(vals, wgx_ref[...], preferred_element_type=f32) + bg_ref[...]
    xu_all = jnp.dot(vals, wux_ref[...], preferred_element_type=f32) + bu_ref[...]
    xr_all = jnp.dot(vals, wrx_ref[...], preferred_element_type=f32) + br_ref[...]
    min_scr[...] = m_in_all.reshape(S, B, M)
    xg_scr[...] = xg_all.reshape(S, B, M)
    xu_scr[...] = xu_all.reshape(S, B, M)
    xr_scr[...] = xr_all.reshape(S, B, M)

    wgh = wgh_ref[...]
    wuh = wuh_ref[...]
    wrh = wrh_ref[...]

    H = B // 2

    def gru_chunk(memC, xrC, xgC, xuC, ww3C):
        mem2 = memC.reshape(H * NS, M)
        reset = jax.nn.sigmoid(
            jnp.dot(mem2, wrh, preferred_element_type=f32).reshape(H, NS, M)
            + xrC[:, None, :])
        upd = jax.nn.sigmoid(
            jnp.dot(mem2, wgh, preferred_element_type=f32).reshape(H, NS, M)
            + xgC[:, None, :])
        rh = (reset * memC).reshape(H * NS, M)
        cand = jnp.tanh(
            jnp.dot(rh, wuh, preferred_element_type=f32).reshape(H, NS, M)
            + xuC[:, None, :])
        updated = memC + (ww3C * upd) * (cand - memC)
        nsq = jnp.sum(updated * updated, axis=2, keepdims=True)
        return updated * jax.lax.rsqrt(jnp.maximum(nsq, 1e-24))

    def step(t, carry):
        memA, memB, usage, age = carry
        m_in = min_scr[t]
        xg = xg_scr[t]
        xu = xu_scr[t]
        xr = xr_scr[t]

        simA = jnp.sum(memA * m_in[:H, None, :], axis=2)
        simB = jnp.sum(memB * m_in[H:, None, :], axis=2)
        sim = jnp.concatenate([simA, simB], axis=0)
        scores = usage * 0.2 + age * 0.1 - sim
        w = scores - jnp.max(scores, axis=1, keepdims=True)
        e = jnp.exp(w)
        write_w = e / jnp.sum(e, axis=1, keepdims=True)
        wwm = jnp.where(write_w > 0.01, write_w, jnp.zeros_like(write_w))
        ww3 = (wwm * _UPDATE_RATE)[:, :, None]

        memnA = gru_chunk(memA, xr[:H], xg[:H], xu[:H], ww3[:H])
        memnB = gru_chunk(memB, xr[H:], xg[H:], xu[H:], ww3[H:])

        usage = (usage + wwm) * 0.99
        age = age * _AGE_FACTOR + 1.0
        return memnA, memnB, usage, age

    zeros = jnp.zeros((B, NS), dtype=f32)
    memA, memB, _, _ = jax.lax.fori_loop(
        0, S, step, (mem0_ref[:B // 2], mem0_ref[B // 2:], zeros, zeros),
        unroll=4)
    out_ref[:B // 2] = memA
    out_ref[B // 2:] = memB


@jax.jit
def kernel(hidden_states, memory0, W_in, b_in, W_val, b_val,
           W_gate, b_gate, W_upd, b_upd, W_reset, b_reset):
    B, S, D = hidden_states.shape
    _, NS, M = memory0.shape

    hs = jnp.transpose(hidden_states, (1, 0, 2)).reshape(S * B, D)
    win_t = W_in.T
    wval_t = W_val.T
    wgx, wgh = W_gate[:, :M].T, W_gate[:, M:].T
    wux, wuh = W_upd[:, :M].T, W_upd[:, M:].T
    wrx, wrh = W_reset[:, :M].T, W_reset[:, M:].T

    body = functools.partial(_body, S, B, NS, M)
    out = pl.pallas_call(
        body,
        out_shape=jax.ShapeDtypeStruct((B, NS, M), jnp.float32),
        scratch_shapes=[pltpu.VMEM((S, B, M), jnp.float32)] * 4,
    )(hs, memory0,
      win_t, wval_t, wgx, wgh, wux, wuh, wrx, wrh,
      b_in.reshape(1, M), b_val.reshape(1, M), b_gate.reshape(1, M),
      b_upd.reshape(1, M), b_reset.reshape(1, M))
    return out
